# baseline (speedup 1.0000x reference)
import functools
import numpy as np
import jax
import jax.numpy as jnp
from jax.experimental import pallas as pl
from jax.experimental.pallas import tpu as pltpu

_DIM = 32
_EPS = 1e-4
_F32 = jnp.float32


def _rnn_kernel(x_ref, iW1_ref, ib1_ref, iW2_ref, ib2_ref, Wi_ref, bi_ref,
                Wh_ref, bh_ref, out_ref, xp_s, h_scr, *, tb, B, H, CH):
    j = pl.program_id(1)

    @pl.when(j == 0)
    def _():
        h_scr[...] = jnp.zeros_like(h_scr)

    rev = pl.program_id(0) == 1
    Wi = Wi_ref[0]
    bi = bi_ref[0]
    Wh = Wh_ref[0]
    bh = bh_ref[0]

    bf16 = jnp.bfloat16
    for c in range(tb * B // CH):
        xc = x_ref[c * CH:(c + 1) * CH, :]
        h1 = jax.nn.relu(
            jnp.dot(xc.astype(bf16), iW1_ref[...], preferred_element_type=_F32)
            + ib1_ref[...])
        u = (jnp.dot(h1.astype(bf16), iW2_ref[...], preferred_element_type=_F32)
             + ib2_ref[...])
        xp_s[c * CH:(c + 1) * CH, :] = (
            jnp.dot(u.astype(bf16), Wi, preferred_element_type=_F32) + bi)

    def step(i, h):
        t = jnp.where(rev, tb - 1 - i, i)
        row = pl.multiple_of(t * B, B)
        xpt = xp_s[pl.ds(row, B), :]
        hW = jnp.dot(h.astype(bf16), Wh,
                     preferred_element_type=_F32)
        rz = jax.nn.sigmoid(xpt[:, :2 * H] + hW[:, :2 * H])
        r = rz[:, :H]
        z = rz[:, H:]
        n = jnp.tanh(xpt[:, 2 * H:] + r * (hW[:, 2 * H:] + bh))
        h_new = (1.0 - z) * n + z * h
        out_ref[0, pl.ds(row, B), :] = h_new
        return h_new

    h_scr[...] = jax.lax.fori_loop(0, tb, step, h_scr[...])


def _heads_kernel(h_ref, mW1a_ref, mW1b_ref, mb1_ref, mW2_ref, mb2_ref,
                  vW1a_ref, vW1b_ref, vb1_ref, vW2d_ref, vb2d_ref,
                  vW2o_ref, vb2o_ref, dW1a_ref, dW1b_ref, db1_ref,
                  dW2_ref, db2_ref, Sdiag_ref, Soff_ref,
                  b_ref, A_ref, L_ref, *, tb, B, d):
    hf = h_ref[0]
    hb = h_ref[1]

    def to_bmajor(v, lastdims):
        return jnp.swapaxes(v.reshape((tb, B) + lastdims), 0, 1)

    def head_hidden(Wa_ref, Wb_ref, b1_ref):
        return jax.nn.relu(
            jnp.dot(hf, Wa_ref[...], preferred_element_type=_F32)
            + jnp.dot(hb, Wb_ref[...], preferred_element_type=_F32)
            + b1_ref[...])

    hm = head_hidden(mW1a_ref, mW1b_ref, mb1_ref)
    bv = jnp.dot(hm, mW2_ref[...], preferred_element_type=_F32) + mb2_ref[...]
    b_ref[...] = to_bmajor(bv, (d,))

    hd = head_hidden(dW1a_ref, dW1b_ref, db1_ref)
    Av = jnp.dot(hd, dW2_ref[...], preferred_element_type=_F32) + db2_ref[...]
    A_ref[...] = to_bmajor(Av, (d * d,))

    hv = head_hidden(vW1a_ref, vW1b_ref, vb1_ref)
    Dg = jnp.dot(hv, vW2d_ref[...], preferred_element_type=_F32) + vb2d_ref[...]
    Ao = jnp.dot(hv, vW2o_ref[...], preferred_element_type=_F32) + vb2o_ref[...]
    De = jnp.exp(Dg) + _EPS
    L_ref[...] = (jnp.dot(Ao, Soff_ref[...], preferred_element_type=_F32)
                  + jnp.dot(De, Sdiag_ref[...], preferred_element_type=_F32))


def _llt_kernel(L_ref, q_ref, *, tb, B, d):
    Lb = L_ref[...]
    q = jax.lax.dot_general(
        Lb, Lb, dimension_numbers=(((2,), (2,)), ((0,), (0,))),
        preferred_element_type=_F32)
    q_ref[...] = jnp.swapaxes(q.reshape(tb, B, d, d), 0, 1)


def kernel(x, iW1, ib1, iW2, ib2, fWi, fbi, fWh, fbhn, bWi, bbi, bWh, bbhn,
           mW1, mb1, mW2, mb2, vW1, vb1, vW2, vb2, dW1, db1, dW2, db2):
    B, T, Din = x.shape
    H = fWh.shape[0]
    Hm = iW1.shape[1]
    d = _DIM
    N = B * T
    nvar = d * (d + 1) // 2

    bf16 = jnp.bfloat16
    xT = jnp.swapaxes(x, 0, 1).reshape(N, Din)
    Wis = jnp.stack([fWi, bWi]).astype(bf16)
    bis = jnp.stack([fbi, bbi])[:, None, :]
    Whs = jnp.stack([fWh, bWh]).astype(bf16)
    bhs = jnp.stack([fbhn, bbhn])[:, None, :]
    iW1b = iW1.astype(bf16)
    iW2b = iW2.astype(bf16)

    rows, cols = np.tril_indices(d, k=-1)
    Soff_np = np.zeros((nvar - d, d * d), np.float32)
    Soff_np[np.arange(nvar - d), rows * d + cols] = 1.0
    Sdiag_np = np.zeros((d, d * d), np.float32)
    Sdiag_np[np.arange(d), np.arange(d) * d + np.arange(d)] = 1.0
    Soff = jnp.asarray(Soff_np)
    Sdiag = jnp.asarray(Sdiag_np)

    row2 = lambda v: v[None, :]

    tb = min(64, T)
    nT = T // tb
    CH = min(256, tb * B)
    h_all = pl.pallas_call(
        functools.partial(_rnn_kernel, tb=tb, B=B, H=H, CH=CH),
        grid=(2, nT),
        in_specs=[
            pl.BlockSpec((tb * B, Din),
                         lambda dd, j: (jnp.where(dd == 0, j, nT - 1 - j), 0)),
            pl.BlockSpec((Din, Hm), lambda dd, j: (0, 0)),
            pl.BlockSpec((1, Hm), lambda dd, j: (0, 0)),
            pl.BlockSpec((Hm, H), lambda dd, j: (0, 0)),
            pl.BlockSpec((1, H), lambda dd, j: (0, 0)),
            pl.BlockSpec((1, H, 3 * H), lambda dd, j: (dd, 0, 0)),
            pl.BlockSpec((1, 1, 3 * H), lambda dd, j: (dd, 0, 0)),
            pl.BlockSpec((1, H, 3 * H), lambda dd, j: (dd, 0, 0)),
            pl.BlockSpec((1, 1, H), lambda dd, j: (dd, 0, 0)),
        ],
        out_specs=pl.BlockSpec(
            (1, tb * B, H),
            lambda dd, j: (dd, jnp.where(dd == 0, j, nT - 1 - j), 0)),
        out_shape=jax.ShapeDtypeStruct((2, N, H), _F32),
        scratch_shapes=[pltpu.VMEM((tb * B, 3 * H), _F32),
                        pltpu.VMEM((B, H), _F32)],
        compiler_params=pltpu.CompilerParams(
            dimension_semantics=("parallel", "arbitrary"),
            vmem_limit_bytes=110 * 1024 * 1024),
    )(xT, iW1b, row2(ib1), iW2b, row2(ib2), Wis, bis, Whs, bhs)

    tb2 = min(T, max(1, 512 // B))
    RC = tb2 * B
    nC = T // tb2
    b_flat, A_flat, L_flat = pl.pallas_call(
        functools.partial(_heads_kernel, tb=tb2, B=B, d=d),
        grid=(nC,),
        in_specs=[
            pl.BlockSpec((2, RC, H), lambda i: (0, i, 0)),
            pl.BlockSpec((H, Hm), lambda i: (0, 0)),
            pl.BlockSpec((H, Hm), lambda i: (0, 0)),
            pl.BlockSpec((1, Hm), lambda i: (0, 0)),
            pl.BlockSpec((Hm, d), lambda i: (0, 0)),
            pl.BlockSpec((1, d), lambda i: (0, 0)),
            pl.BlockSpec((H, Hm), lambda i: (0, 0)),
            pl.BlockSpec((H, Hm), lambda i: (0, 0)),
            pl.BlockSpec((1, Hm), lambda i: (0, 0)),
            pl.BlockSpec((Hm, d), lambda i: (0, 0)),
            pl.BlockSpec((1, d), lambda i: (0, 0)),
            pl.BlockSpec((Hm, nvar - d), lambda i: (0, 0)),
            pl.BlockSpec((1, nvar - d), lambda i: (0, 0)),
            pl.BlockSpec((H, Hm), lambda i: (0, 0)),
            pl.BlockSpec((H, Hm), lambda i: (0, 0)),
            pl.BlockSpec((1, Hm), lambda i: (0, 0)),
            pl.BlockSpec((Hm, d * d), lambda i: (0, 0)),
            pl.BlockSpec((1, d * d), lambda i: (0, 0)),
            pl.BlockSpec((d, d * d), lambda i: (0, 0)),
            pl.BlockSpec((nvar - d, d * d), lambda i: (0, 0)),
        ],
        out_specs=[
            pl.BlockSpec((B, tb2, d), lambda i: (0, i, 0)),
            pl.BlockSpec((B, tb2, d * d), lambda i: (0, i, 0)),
            pl.BlockSpec((RC, d * d), lambda i: (i, 0)),
        ],
        out_shape=[
            jax.ShapeDtypeStruct((B, T, d), _F32),
            jax.ShapeDtypeStruct((B, T, d * d), _F32),
            jax.ShapeDtypeStruct((N, d * d), _F32),
        ],
        compiler_params=pltpu.CompilerParams(
            dimension_semantics=("parallel",),
            vmem_limit_bytes=110 * 1024 * 1024),
    )(h_all,
      mW1[:H], mW1[H:], row2(mb1), mW2, row2(mb2),
      vW1[:H], vW1[H:], row2(vb1), vW2[:, :d], row2(vb2[:d]),
      vW2[:, d:], row2(vb2[d:]),
      dW1[:H], dW1[H:], row2(db1), dW2, row2(db2),
      Sdiag, Soff)

    tbq = min(T, max(1, 512 // B))
    RD = tbq * B
    L3 = L_flat.reshape(N, d, d)
    Q_out = pl.pallas_call(
        functools.partial(_llt_kernel, tb=tbq, B=B, d=d),
        grid=(T // tbq,),
        in_specs=[pl.BlockSpec((RD, d, d), lambda i: (i, 0, 0))],
        out_specs=pl.BlockSpec((B, tbq, d, d), lambda i: (0, i, 0, 0)),
        out_shape=jax.ShapeDtypeStruct((B, T, d, d), _F32),
        compiler_params=pltpu.CompilerParams(
            dimension_semantics=("parallel",),
            vmem_limit_bytes=110 * 1024 * 1024),
    )(L3)

    A = A_flat.reshape(B, T, d, d)
    return A, b_flat, Q_out


# speedup vs baseline: 1.0040x; 1.0040x over previous
import functools
import numpy as np
import jax
import jax.numpy as jnp
from jax.experimental import pallas as pl
from jax.experimental.pallas import tpu as pltpu

_DIM = 32
_EPS = 1e-4
_F32 = jnp.float32


def _rnn_kernel(x_ref, iW1_ref, ib1_ref, iW2_ref, ib2_ref, Wi_ref, bi_ref,
                Wh_ref, bh_ref, out_ref, xp_s, h_scr, *, tb, B, H, CH):
    j = pl.program_id(1)

    @pl.when(j == 0)
    def _():
        h_scr[...] = jnp.zeros_like(h_scr)

    rev = pl.program_id(0) == 1
    Wi = Wi_ref[0]
    bi = bi_ref[0]
    Wh = Wh_ref[0]
    bh = bh_ref[0]

    for c in range(tb * B // CH):
        xc = x_ref[c * CH:(c + 1) * CH, :]
        h1 = jax.nn.relu(
            jnp.dot(xc, iW1_ref[...], preferred_element_type=_F32)
            + ib1_ref[...])
        u = jnp.dot(h1, iW2_ref[...], preferred_element_type=_F32) + ib2_ref[...]
        xp_s[c * CH:(c + 1) * CH, :] = (
            jnp.dot(u, Wi, preferred_element_type=_F32) + bi)

    def step(i, h):
        t = jnp.where(rev, tb - 1 - i, i)
        row = pl.multiple_of(t * B, B)
        xpt = xp_s[pl.ds(row, B), :]
        hW = jnp.dot(h, Wh, preferred_element_type=_F32)
        rz = jax.nn.sigmoid(xpt[:, :2 * H] + hW[:, :2 * H])
        r = rz[:, :H]
        z = rz[:, H:]
        n = jnp.tanh(xpt[:, 2 * H:] + r * (hW[:, 2 * H:] + bh))
        h_new = (1.0 - z) * n + z * h
        out_ref[0, pl.ds(row, B), :] = h_new
        return h_new

    h_scr[...] = jax.lax.fori_loop(0, tb, step, h_scr[...])


def _heads_kernel(h_ref, mW1a_ref, mW1b_ref, mb1_ref, mW2_ref, mb2_ref,
                  vW1a_ref, vW1b_ref, vb1_ref, vW2d_ref, vb2d_ref,
                  vW2o_ref, vb2o_ref, dW1a_ref, dW1b_ref, db1_ref,
                  dW2_ref, db2_ref, Sdiag_ref, Soff_ref,
                  b_ref, A_ref, L_ref, *, tb, B, d):
    hf = h_ref[0]
    hb = h_ref[1]

    def to_bmajor(v, lastdims):
        return jnp.swapaxes(v.reshape((tb, B) + lastdims), 0, 1)

    def head_hidden(Wa_ref, Wb_ref, b1_ref):
        return jax.nn.relu(
            jnp.dot(hf, Wa_ref[...], preferred_element_type=_F32)
            + jnp.dot(hb, Wb_ref[...], preferred_element_type=_F32)
            + b1_ref[...])

    hm = head_hidden(mW1a_ref, mW1b_ref, mb1_ref)
    bv = jnp.dot(hm, mW2_ref[...], preferred_element_type=_F32) + mb2_ref[...]
    b_ref[...] = to_bmajor(bv, (d,))

    hd = head_hidden(dW1a_ref, dW1b_ref, db1_ref)
    Av = jnp.dot(hd, dW2_ref[...], preferred_element_type=_F32) + db2_ref[...]
    A_ref[...] = to_bmajor(Av, (d * d,))

    hv = head_hidden(vW1a_ref, vW1b_ref, vb1_ref)
    Dg = jnp.dot(hv, vW2d_ref[...], preferred_element_type=_F32) + vb2d_ref[...]
    Ao = jnp.dot(hv, vW2o_ref[...], preferred_element_type=_F32) + vb2o_ref[...]
    De = jnp.exp(Dg) + _EPS
    L_ref[...] = (jnp.dot(Ao, Soff_ref[...], preferred_element_type=_F32)
                  + jnp.dot(De, Sdiag_ref[...], preferred_element_type=_F32))


def _llt_kernel(L_ref, q_ref, *, tb, B, d):
    Lb = L_ref[...]
    q = jax.lax.dot_general(
        Lb, Lb, dimension_numbers=(((2,), (2,)), ((0,), (0,))),
        preferred_element_type=_F32)
    q_ref[...] = jnp.swapaxes(q.reshape(tb, B, d, d), 0, 1)


def kernel(x, iW1, ib1, iW2, ib2, fWi, fbi, fWh, fbhn, bWi, bbi, bWh, bbhn,
           mW1, mb1, mW2, mb2, vW1, vb1, vW2, vb2, dW1, db1, dW2, db2):
    B, T, Din = x.shape
    H = fWh.shape[0]
    Hm = iW1.shape[1]
    d = _DIM
    N = B * T
    nvar = d * (d + 1) // 2

    xT = jnp.swapaxes(x, 0, 1).reshape(N, Din)
    Wis = jnp.stack([fWi, bWi])
    bis = jnp.stack([fbi, bbi])[:, None, :]
    Whs = jnp.stack([fWh, bWh])
    bhs = jnp.stack([fbhn, bbhn])[:, None, :]

    rows, cols = np.tril_indices(d, k=-1)
    Soff_np = np.zeros((nvar - d, d * d), np.float32)
    Soff_np[np.arange(nvar - d), rows * d + cols] = 1.0
    Sdiag_np = np.zeros((d, d * d), np.float32)
    Sdiag_np[np.arange(d), np.arange(d) * d + np.arange(d)] = 1.0
    Soff = jnp.asarray(Soff_np)
    Sdiag = jnp.asarray(Sdiag_np)

    row2 = lambda v: v[None, :]

    tb = min(64, T)
    nT = T // tb
    CH = min(256, tb * B)
    h_all = pl.pallas_call(
        functools.partial(_rnn_kernel, tb=tb, B=B, H=H, CH=CH),
        grid=(2, nT),
        in_specs=[
            pl.BlockSpec((tb * B, Din),
                         lambda dd, j: (jnp.where(dd == 0, j, nT - 1 - j), 0)),
            pl.BlockSpec((Din, Hm), lambda dd, j: (0, 0)),
            pl.BlockSpec((1, Hm), lambda dd, j: (0, 0)),
            pl.BlockSpec((Hm, H), lambda dd, j: (0, 0)),
            pl.BlockSpec((1, H), lambda dd, j: (0, 0)),
            pl.BlockSpec((1, H, 3 * H), lambda dd, j: (dd, 0, 0)),
            pl.BlockSpec((1, 1, 3 * H), lambda dd, j: (dd, 0, 0)),
            pl.BlockSpec((1, H, 3 * H), lambda dd, j: (dd, 0, 0)),
            pl.BlockSpec((1, 1, H), lambda dd, j: (dd, 0, 0)),
        ],
        out_specs=pl.BlockSpec(
            (1, tb * B, H),
            lambda dd, j: (dd, jnp.where(dd == 0, j, nT - 1 - j), 0)),
        out_shape=jax.ShapeDtypeStruct((2, N, H), _F32),
        scratch_shapes=[pltpu.VMEM((tb * B, 3 * H), _F32),
                        pltpu.VMEM((B, H), _F32)],
        compiler_params=pltpu.CompilerParams(
            dimension_semantics=("parallel", "arbitrary"),
            vmem_limit_bytes=110 * 1024 * 1024),
    )(xT, iW1, row2(ib1), iW2, row2(ib2), Wis, bis, Whs, bhs)

    tb2 = min(T, max(1, 512 // B))
    RC = tb2 * B
    nC = T // tb2
    b_flat, A_flat, L_flat = pl.pallas_call(
        functools.partial(_heads_kernel, tb=tb2, B=B, d=d),
        grid=(nC,),
        in_specs=[
            pl.BlockSpec((2, RC, H), lambda i: (0, i, 0)),
            pl.BlockSpec((H, Hm), lambda i: (0, 0)),
            pl.BlockSpec((H, Hm), lambda i: (0, 0)),
            pl.BlockSpec((1, Hm), lambda i: (0, 0)),
            pl.BlockSpec((Hm, d), lambda i: (0, 0)),
            pl.BlockSpec((1, d), lambda i: (0, 0)),
            pl.BlockSpec((H, Hm), lambda i: (0, 0)),
            pl.BlockSpec((H, Hm), lambda i: (0, 0)),
            pl.BlockSpec((1, Hm), lambda i: (0, 0)),
            pl.BlockSpec((Hm, d), lambda i: (0, 0)),
            pl.BlockSpec((1, d), lambda i: (0, 0)),
            pl.BlockSpec((Hm, nvar - d), lambda i: (0, 0)),
            pl.BlockSpec((1, nvar - d), lambda i: (0, 0)),
            pl.BlockSpec((H, Hm), lambda i: (0, 0)),
            pl.BlockSpec((H, Hm), lambda i: (0, 0)),
            pl.BlockSpec((1, Hm), lambda i: (0, 0)),
            pl.BlockSpec((Hm, d * d), lambda i: (0, 0)),
            pl.BlockSpec((1, d * d), lambda i: (0, 0)),
            pl.BlockSpec((d, d * d), lambda i: (0, 0)),
            pl.BlockSpec((nvar - d, d * d), lambda i: (0, 0)),
        ],
        out_specs=[
            pl.BlockSpec((B, tb2, d), lambda i: (0, i, 0)),
            pl.BlockSpec((B, tb2, d * d), lambda i: (0, i, 0)),
            pl.BlockSpec((RC, d * d), lambda i: (i, 0)),
        ],
        out_shape=[
            jax.ShapeDtypeStruct((B, T, d), _F32),
            jax.ShapeDtypeStruct((B, T, d * d), _F32),
            jax.ShapeDtypeStruct((N, d * d), _F32),
        ],
        compiler_params=pltpu.CompilerParams(
            dimension_semantics=("parallel",),
            vmem_limit_bytes=110 * 1024 * 1024),
    )(h_all,
      mW1[:H], mW1[H:], row2(mb1), mW2, row2(mb2),
      vW1[:H], vW1[H:], row2(vb1), vW2[:, :d], row2(vb2[:d]),
      vW2[:, d:], row2(vb2[d:]),
      dW1[:H], dW1[H:], row2(db1), dW2, row2(db2),
      Sdiag, Soff)

    tbq = min(T, max(1, 512 // B))
    RD = tbq * B
    L3 = L_flat.reshape(N, d, d)
    Q_out = pl.pallas_call(
        functools.partial(_llt_kernel, tb=tbq, B=B, d=d),
        grid=(T // tbq,),
        in_specs=[pl.BlockSpec((RD, d, d), lambda i: (i, 0, 0))],
        out_specs=pl.BlockSpec((B, tbq, d, d), lambda i: (0, i, 0, 0)),
        out_shape=jax.ShapeDtypeStruct((B, T, d, d), _F32),
        compiler_params=pltpu.CompilerParams(
            dimension_semantics=("parallel",),
            vmem_limit_bytes=110 * 1024 * 1024),
    )(L3)

    A = A_flat.reshape(B, T, d, d)
    return A, b_flat, Q_out
